# Initial kernel scaffold; baseline (speedup 1.0000x reference)
import jax
import jax.numpy as jnp
from jax.experimental import pallas as pl


def kernel(z, W):
    raise NotImplementedError("write your pallas kernel here")


# speedup vs baseline: 1.4565x; 1.4565x over previous
import jax
import jax.numpy as jnp
from jax.experimental import pallas as pl
from jax.experimental.pallas import tpu as pltpu

_B, _D, _L, _E = 64, 64, 1024, 1024
_N = _B * _D * _L


def _vq_body(z_ref, w_ref, q_ref, loss_ref):
    b = pl.program_id(0)
    z = z_ref[0]
    w = w_ref[...]
    s = jax.lax.dot_general(
        w, z, (((1,), (0,)), ((), ())), preferred_element_type=jnp.float32
    )
    zsq = jnp.sum(z * z, axis=0, keepdims=True)
    wsq = jnp.sum(w * w, axis=1, keepdims=True)
    d = (zsq + wsq) - 2.0 * s
    dmin = jnp.min(d, axis=0, keepdims=True)
    eiota = jax.lax.broadcasted_iota(jnp.int32, (_E, _L), 0)
    cand = jnp.where(d == dmin, eiota, _E)
    idx = jnp.min(cand, axis=0, keepdims=True)
    onehot = (eiota == idx).astype(jnp.float32)
    q = jax.lax.dot_general(
        w, onehot, (((0,), (0,)), ((), ())),
        preferred_element_type=jnp.float32,
        precision=jax.lax.Precision.HIGHEST,
    )
    q_ref[0] = q
    diff = q - z
    part = jnp.sum(diff * diff).reshape(1, 1)

    @pl.when(b == 0)
    def _init():
        loss_ref[...] = jnp.zeros((1, 1), jnp.float32)

    loss_ref[...] += part


@jax.jit
def kernel(z, W):
    q, loss_sum = pl.pallas_call(
        _vq_body,
        grid=(_B,),
        in_specs=[
            pl.BlockSpec((1, _D, _L), lambda b: (b, 0, 0)),
            pl.BlockSpec((_E, _D), lambda b: (0, 0)),
        ],
        out_specs=[
            pl.BlockSpec((1, _D, _L), lambda b: (b, 0, 0)),
            pl.BlockSpec((1, 1), lambda b: (0, 0)),
        ],
        out_shape=[
            jax.ShapeDtypeStruct((_B, _D, _L), jnp.float32),
            jax.ShapeDtypeStruct((1, 1), jnp.float32),
        ],
    )(z, W)
    vq_loss = loss_sum[0, 0] / _N
    return q, vq_loss, 0.25 * vq_loss


# speedup vs baseline: 2.8726x; 1.9722x over previous
import jax
import jax.numpy as jnp
from jax.experimental import pallas as pl
from jax.experimental.pallas import tpu as pltpu

_B, _D, _L, _E = 64, 64, 1024, 1024
_N = _B * _D * _L


def _vq_body(z_ref, w_ref, q_ref, loss_ref):
    b = pl.program_id(0)
    z = z_ref[0]
    w = w_ref[...]
    s = jax.lax.dot_general(
        w, z, (((1,), (0,)), ((), ())), preferred_element_type=jnp.float32
    )
    zsq = jnp.sum(z * z, axis=0, keepdims=True)
    wsq = jnp.sum(w * w, axis=1, keepdims=True)
    d = (zsq + wsq) - 2.0 * s
    dmin = jnp.min(d, axis=0, keepdims=True)
    eiota = jax.lax.broadcasted_iota(jnp.int32, (_E, _L), 0)
    cand = jnp.where(d == dmin, eiota, _E)
    idx = jnp.min(cand, axis=0, keepdims=True)
    onehot = (eiota == idx).astype(jnp.float32)
    q = jax.lax.dot_general(
        w, onehot, (((0,), (0,)), ((), ())),
        preferred_element_type=jnp.float32,
    )
    q_ref[0] = q
    diff = q - z
    part = jnp.sum(diff * diff).reshape(1, 1)

    @pl.when(b == 0)
    def _init():
        loss_ref[...] = jnp.zeros((1, 1), jnp.float32)

    loss_ref[...] += part


@jax.jit
def kernel(z, W):
    q, loss_sum = pl.pallas_call(
        _vq_body,
        grid=(_B,),
        in_specs=[
            pl.BlockSpec((1, _D, _L), lambda b: (b, 0, 0)),
            pl.BlockSpec((_E, _D), lambda b: (0, 0)),
        ],
        out_specs=[
            pl.BlockSpec((1, _D, _L), lambda b: (b, 0, 0)),
            pl.BlockSpec((1, 1), lambda b: (0, 0)),
        ],
        out_shape=[
            jax.ShapeDtypeStruct((_B, _D, _L), jnp.float32),
            jax.ShapeDtypeStruct((1, 1), jnp.float32),
        ],
    )(z, W)
    vq_loss = loss_sum[0, 0] / _N
    return q, vq_loss, 0.25 * vq_loss


# speedup vs baseline: 3.1317x; 1.0902x over previous
import jax
import jax.numpy as jnp
from jax.experimental import pallas as pl
from jax.experimental.pallas import tpu as pltpu

_B, _D, _L, _E = 64, 64, 1024, 1024
_N = _B * _D * _L


def _vq_body(z_ref, w_ref, q_ref, loss_ref):
    b = pl.program_id(0)
    z = z_ref[0]
    w = w_ref[...]
    s2 = jax.lax.dot_general(
        -2.0 * w, z, (((1,), (0,)), ((), ())), preferred_element_type=jnp.float32
    )
    zsq = jnp.sum(z * z, axis=0, keepdims=True)
    wsq = jnp.sum(w * w, axis=1, keepdims=True)
    d = (zsq + wsq) + s2
    dmin = jnp.min(d, axis=0, keepdims=True)
    eiota = jax.lax.broadcasted_iota(jnp.int32, (_E, _L), 0)
    cand = jnp.where(d == dmin, eiota, _E)
    idx = jnp.min(cand, axis=0, keepdims=True)
    onehot = (eiota == idx).astype(jnp.float32)
    q = jax.lax.dot_general(
        w, onehot, (((0,), (0,)), ((), ())),
        preferred_element_type=jnp.float32,
    )
    q_ref[0] = q
    part = jnp.sum(dmin).reshape(1, 1)

    @pl.when(b == 0)
    def _init():
        loss_ref[...] = jnp.zeros((1, 1), jnp.float32)

    loss_ref[...] += part


@jax.jit
def kernel(z, W):
    q, loss_sum = pl.pallas_call(
        _vq_body,
        grid=(_B,),
        in_specs=[
            pl.BlockSpec((1, _D, _L), lambda b: (b, 0, 0)),
            pl.BlockSpec((_E, _D), lambda b: (0, 0)),
        ],
        out_specs=[
            pl.BlockSpec((1, _D, _L), lambda b: (b, 0, 0)),
            pl.BlockSpec((1, 1), lambda b: (0, 0)),
        ],
        out_shape=[
            jax.ShapeDtypeStruct((_B, _D, _L), jnp.float32),
            jax.ShapeDtypeStruct((1, 1), jnp.float32),
        ],
    )(z, W)
    vq_loss = loss_sum[0, 0] / _N
    return q, vq_loss, 0.25 * vq_loss


# speedup vs baseline: 3.8392x; 1.2259x over previous
import jax
import jax.numpy as jnp
from jax.experimental import pallas as pl
from jax.experimental.pallas import tpu as pltpu

_B, _D, _L, _E = 64, 64, 1024, 1024
_N = _B * _D * _L


def _vq_body(z_ref, w_ref, q_ref, loss_ref):
    b = pl.program_id(0)
    z = z_ref[0]
    w = w_ref[...]
    s2 = jax.lax.dot_general(
        -2.0 * w, z, (((1,), (0,)), ((), ())), preferred_element_type=jnp.float32
    )
    zsq = jnp.sum(z * z, axis=0, keepdims=True)
    wsq = jnp.sum(w * w, axis=1, keepdims=True)
    d = (zsq + wsq) + s2
    dmin = jnp.min(d, axis=0, keepdims=True)
    eiota = jax.lax.broadcasted_iota(jnp.int32, (_E, _L), 0)
    idx = jnp.argmin(d, axis=0).reshape(1, _L)
    onehot = (eiota == idx).astype(jnp.float32)
    q = jax.lax.dot_general(
        w, onehot, (((0,), (0,)), ((), ())),
        preferred_element_type=jnp.float32,
    )
    q_ref[0] = q
    part = jnp.sum(dmin).reshape(1, 1)

    @pl.when(b == 0)
    def _init():
        loss_ref[...] = jnp.zeros((1, 1), jnp.float32)

    loss_ref[...] += part


@jax.jit
def kernel(z, W):
    q, loss_sum = pl.pallas_call(
        _vq_body,
        grid=(_B,),
        in_specs=[
            pl.BlockSpec((1, _D, _L), lambda b: (b, 0, 0)),
            pl.BlockSpec((_E, _D), lambda b: (0, 0)),
        ],
        out_specs=[
            pl.BlockSpec((1, _D, _L), lambda b: (b, 0, 0)),
            pl.BlockSpec((1, 1), lambda b: (0, 0)),
        ],
        out_shape=[
            jax.ShapeDtypeStruct((_B, _D, _L), jnp.float32),
            jax.ShapeDtypeStruct((1, 1), jnp.float32),
        ],
    )(z, W)
    vq_loss = loss_sum[0, 0] / _N
    return q, vq_loss, 0.25 * vq_loss
